# baseline (speedup 1.0000x reference)
import functools

import jax
import jax.numpy as jnp
from jax import lax
from jax.experimental import pallas as pl
from jax.experimental.pallas import tpu as pltpu
from jax.experimental.pallas import tpu_sc as plsc

B = 16384
NUM = 13
NF = 26
V = 100000
OUT = 1

NC = 2
NS = 16
NW = NC * NS
L = 16

BW = B // NW
CHUNK = 128
VPR = BW // L
RPC = BW // CHUNK

GROUPS = (4, 6, 8, 8)


def _make_body(nf, with_dense):
    nchunk = (nf * BW) // CHUNK

    def _body(cats3, w_b, bias_b, ei_b, cp_flat, dummy, nums3, out_hbm,
              idx_v, vals_v, nums_v, w_v, b_v, ei_v, out_v, gsem):
        wid = lax.axis_index("s") * NC + lax.axis_index("c")

        pltpu.sync_copy(cats3.at[wid], idx_v)
        pltpu.sync_copy(ei_b, ei_v)

        def add_off(r, c):
            off = ei_v[r // RPC, :]
            for q in range(CHUNK // L):
                idx_v[r, pl.ds(q * L, L)] += off
            return c
        lax.fori_loop(0, nchunk, add_off, 0)

        def fire(j, c):
            pltpu.async_copy(cp_flat.at[idx_v.at[j]], vals_v.at[j], gsem)
            return c
        lax.fori_loop(0, nchunk, fire, 0)

        if with_dense:
            pltpu.sync_copy(nums3.at[wid], nums_v)
            pltpu.sync_copy(w_b, w_v)
            pltpu.sync_copy(bias_b, b_v)
            wrows = [w_v[j, :] for j in range(NUM)]
            bvec = b_v[...]

            def dense(i, c):
                acc = bvec
                for j in range(NUM):
                    acc = acc + nums_v[j, pl.ds(i * L, L)] * wrows[j]
                out_v[pl.ds(i * L, L)] = acc
                return c
            lax.fori_loop(0, VPR, dense, 0)

        pltpu.make_async_copy(dummy, vals_v, gsem).wait()

        def reduce(i, c):
            row0 = i // (CHUNK // L)
            col = (i % (CHUNK // L)) * L
            if with_dense:
                acc = out_v[pl.ds(i * L, L)]
                f0 = 0
            else:
                acc = vals_v[row0, pl.ds(col, L)]
                f0 = 1
            for f in range(f0, nf):
                acc = acc + vals_v[f * RPC + row0, pl.ds(col, L)]
            out_v[pl.ds(i * L, L)] = acc
            return c
        lax.fori_loop(0, VPR, reduce, 0)

        pltpu.sync_copy(out_v, out_hbm.at[pl.ds(wid * BW, BW)])
    return _body


def _make_call(nf, with_dense):
    nchunk = (nf * BW) // CHUNK
    return functools.partial(
        pl.kernel,
        out_type=jax.ShapeDtypeStruct((B,), jnp.float32),
        mesh=plsc.VectorSubcoreMesh(core_axis_name="c", subcore_axis_name="s",
                                    num_cores=NC, num_subcores=NS),
        scratch_types=[
            pltpu.VMEM((nchunk, CHUNK), jnp.int32),
            pltpu.VMEM((nchunk, CHUNK), jnp.float32),
            pltpu.VMEM((NUM, BW), jnp.float32),
            pltpu.VMEM((NUM, L), jnp.float32),
            pltpu.VMEM((L,), jnp.float32),
            pltpu.VMEM((nf, L), jnp.int32),
            pltpu.VMEM((BW,), jnp.float32),
            pltpu.SemaphoreType.DMA,
        ],
        compiler_params=pltpu.CompilerParams(use_tc_tiling_on_sc=False),
    )(_make_body(nf, with_dense))


_sc_calls = tuple(_make_call(nf, g == 0) for g, nf in enumerate(GROUPS))


def _group_prep(categories, embed_idx, f0, nf):
    cats = categories[:, f0:f0 + nf]
    cats3 = cats.T.reshape(nf, NW, BW).transpose(1, 0, 2)
    cats3 = cats3.reshape(NW, (nf * BW) // CHUNK, CHUNK)
    ei = embed_idx[f0:f0 + nf].astype(jnp.int32) - jnp.int32(f0 * V)
    ei_b = jnp.broadcast_to(ei.reshape(nf, 1), (nf, L))
    return cats3, ei_b


@jax.jit
def kernel(numbers, categories, W, bias, cat_params, embed_idx):
    nums3 = numbers.T.reshape(NUM, NW, BW).transpose(1, 0, 2)
    w_b = jnp.broadcast_to(W.reshape(NUM, 1), (NUM, L))
    bias_b = jnp.broadcast_to(bias.reshape(1, 1), (1, L)).reshape(L)
    acc = None
    f0 = 0
    for g, nf in enumerate(GROUPS):
        cats3, ei_b = _group_prep(categories, embed_idx, f0, nf)
        flat = cat_params[f0 * V:(f0 + nf) * V].reshape(nf * V)
        dummy = jnp.zeros(((nf * BW) // CHUNK, CHUNK), jnp.float32)
        p = _sc_calls[g](cats3, w_b, bias_b, ei_b, flat, dummy, nums3)
        acc = p if acc is None else acc + p
        f0 += nf
    return acc.reshape(B, OUT)


# speedup vs baseline: 1.1920x; 1.1920x over previous
import functools

import jax
import jax.numpy as jnp
from jax import lax
from jax.experimental import pallas as pl
from jax.experimental.pallas import tpu as pltpu
from jax.experimental.pallas import tpu_sc as plsc

B = 16384
NUM = 13
NF = 26
V = 100000
OUT = 1

NC = 2
NS = 16
NW = NC * NS
L = 16

BW = B // NW
CHUNK = 128
VPR = BW // L
RPC = BW // CHUNK

GROUPS = (6, 5, 5, 5, 5)


def _make_body(nf, with_dense):
    nchunk = (nf * BW) // CHUNK

    def _body(cats3, w_b, bias_b, ei_b, cp_flat, dummy, nums3, out_hbm,
              idx_v, vals_v, nums_v, w_v, b_v, ei_v, out_v, gsem):
        wid = lax.axis_index("s") * NC + lax.axis_index("c")

        pltpu.sync_copy(cats3.at[wid], idx_v)
        pltpu.sync_copy(ei_b, ei_v)

        def add_off(r, c):
            off = ei_v[r // RPC, :]
            for q in range(CHUNK // L):
                idx_v[r, pl.ds(q * L, L)] += off
            return c
        lax.fori_loop(0, nchunk, add_off, 0)

        def fire(j, c):
            pltpu.async_copy(cp_flat.at[idx_v.at[j]], vals_v.at[j], gsem)
            return c
        lax.fori_loop(0, nchunk, fire, 0)

        if with_dense:
            pltpu.sync_copy(nums3.at[wid], nums_v)
            pltpu.sync_copy(w_b, w_v)
            pltpu.sync_copy(bias_b, b_v)
            wrows = [w_v[j, :] for j in range(NUM)]
            bvec = b_v[...]

            def dense(i, c):
                acc = bvec
                for j in range(NUM):
                    acc = acc + nums_v[j, pl.ds(i * L, L)] * wrows[j]
                out_v[pl.ds(i * L, L)] = acc
                return c
            lax.fori_loop(0, VPR, dense, 0)

        pltpu.make_async_copy(dummy, vals_v, gsem).wait()

        def reduce(i, c):
            row0 = i // (CHUNK // L)
            col = (i % (CHUNK // L)) * L
            if with_dense:
                acc = out_v[pl.ds(i * L, L)]
                f0 = 0
            else:
                acc = vals_v[row0, pl.ds(col, L)]
                f0 = 1
            for f in range(f0, nf):
                acc = acc + vals_v[f * RPC + row0, pl.ds(col, L)]
            out_v[pl.ds(i * L, L)] = acc
            return c
        lax.fori_loop(0, VPR, reduce, 0)

        pltpu.sync_copy(out_v, out_hbm.at[pl.ds(wid * BW, BW)])
    return _body


def _make_call(nf, with_dense):
    nchunk = (nf * BW) // CHUNK
    return functools.partial(
        pl.kernel,
        out_type=jax.ShapeDtypeStruct((B,), jnp.float32),
        mesh=plsc.VectorSubcoreMesh(core_axis_name="c", subcore_axis_name="s",
                                    num_cores=NC, num_subcores=NS),
        scratch_types=[
            pltpu.VMEM((nchunk, CHUNK), jnp.int32),
            pltpu.VMEM((nchunk, CHUNK), jnp.float32),
            pltpu.VMEM((NUM, BW), jnp.float32),
            pltpu.VMEM((NUM, L), jnp.float32),
            pltpu.VMEM((L,), jnp.float32),
            pltpu.VMEM((nf, L), jnp.int32),
            pltpu.VMEM((BW,), jnp.float32),
            pltpu.SemaphoreType.DMA,
        ],
        compiler_params=pltpu.CompilerParams(use_tc_tiling_on_sc=False),
    )(_make_body(nf, with_dense))


_sc_calls = tuple(_make_call(nf, g == 0) for g, nf in enumerate(GROUPS))


def _group_prep(categories, embed_idx, f0, nf):
    cats = categories[:, f0:f0 + nf]
    cats3 = cats.T.reshape(nf, NW, BW).transpose(1, 0, 2)
    cats3 = cats3.reshape(NW, (nf * BW) // CHUNK, CHUNK)
    ei = embed_idx[f0:f0 + nf].astype(jnp.int32) - jnp.int32(f0 * V)
    ei_b = jnp.broadcast_to(ei.reshape(nf, 1), (nf, L))
    return cats3, ei_b


@jax.jit
def kernel(numbers, categories, W, bias, cat_params, embed_idx):
    nums3 = numbers.T.reshape(NUM, NW, BW).transpose(1, 0, 2)
    w_b = jnp.broadcast_to(W.reshape(NUM, 1), (NUM, L))
    bias_b = jnp.broadcast_to(bias.reshape(1, 1), (1, L)).reshape(L)
    acc = None
    f0 = 0
    for g, nf in enumerate(GROUPS):
        cats3, ei_b = _group_prep(categories, embed_idx, f0, nf)
        flat = cat_params[f0 * V:(f0 + nf) * V].reshape(nf * V)
        dummy = jnp.zeros(((nf * BW) // CHUNK, CHUNK), jnp.float32)
        p = _sc_calls[g](cats3, w_b, bias_b, ei_b, flat, dummy, nums3)
        acc = p if acc is None else acc + p
        f0 += nf
    return acc.reshape(B, OUT)


# speedup vs baseline: 1.2779x; 1.0721x over previous
import functools

import jax
import jax.numpy as jnp
from jax import lax
from jax.experimental import pallas as pl
from jax.experimental.pallas import tpu as pltpu
from jax.experimental.pallas import tpu_sc as plsc

B = 16384
NUM = 13
NF = 26
V = 100000
OUT = 1

NC = 2
NS = 16
NW = NC * NS
L = 16

BW = B // NW
CHUNK = 128
VPR = BW // L
RPC = BW // CHUNK

GROUPS = (6, 7, 7, 6)


def _make_body(nf, with_dense):
    nchunk = (nf * BW) // CHUNK

    def _body(cats3, w_b, bias_b, ei_b, cp_flat, dummy, nums3, out_hbm,
              idx_v, vals_v, nums_v, w_v, b_v, ei_v, out_v, gsem):
        wid = lax.axis_index("s") * NC + lax.axis_index("c")

        pltpu.sync_copy(cats3.at[wid], idx_v)
        pltpu.sync_copy(ei_b, ei_v)

        def add_off(r, c):
            off = ei_v[r // RPC, :]
            for q in range(CHUNK // L):
                idx_v[r, pl.ds(q * L, L)] += off
            return c
        lax.fori_loop(0, nchunk, add_off, 0)

        def fire(j, c):
            pltpu.async_copy(cp_flat.at[idx_v.at[j]], vals_v.at[j], gsem)
            return c
        lax.fori_loop(0, nchunk, fire, 0)

        if with_dense:
            pltpu.sync_copy(nums3.at[wid], nums_v)
            pltpu.sync_copy(w_b, w_v)
            pltpu.sync_copy(bias_b, b_v)
            wrows = [w_v[j, :] for j in range(NUM)]
            bvec = b_v[...]

            def dense(i, c):
                acc = bvec
                for j in range(NUM):
                    acc = acc + nums_v[j, pl.ds(i * L, L)] * wrows[j]
                out_v[pl.ds(i * L, L)] = acc
                return c
            lax.fori_loop(0, VPR, dense, 0)

        pltpu.make_async_copy(dummy, vals_v, gsem).wait()

        def reduce(i, c):
            row0 = i // (CHUNK // L)
            col = (i % (CHUNK // L)) * L
            if with_dense:
                acc = out_v[pl.ds(i * L, L)]
                f0 = 0
            else:
                acc = vals_v[row0, pl.ds(col, L)]
                f0 = 1
            for f in range(f0, nf):
                acc = acc + vals_v[f * RPC + row0, pl.ds(col, L)]
            out_v[pl.ds(i * L, L)] = acc
            return c
        lax.fori_loop(0, VPR, reduce, 0)

        pltpu.sync_copy(out_v, out_hbm.at[pl.ds(wid * BW, BW)])
    return _body


def _make_call(nf, with_dense):
    nchunk = (nf * BW) // CHUNK
    return functools.partial(
        pl.kernel,
        out_type=jax.ShapeDtypeStruct((B,), jnp.float32),
        mesh=plsc.VectorSubcoreMesh(core_axis_name="c", subcore_axis_name="s",
                                    num_cores=NC, num_subcores=NS),
        scratch_types=[
            pltpu.VMEM((nchunk, CHUNK), jnp.int32),
            pltpu.VMEM((nchunk, CHUNK), jnp.float32),
            pltpu.VMEM((NUM, BW), jnp.float32),
            pltpu.VMEM((NUM, L), jnp.float32),
            pltpu.VMEM((L,), jnp.float32),
            pltpu.VMEM((nf, L), jnp.int32),
            pltpu.VMEM((BW,), jnp.float32),
            pltpu.SemaphoreType.DMA,
        ],
        compiler_params=pltpu.CompilerParams(use_tc_tiling_on_sc=False),
    )(_make_body(nf, with_dense))


_sc_calls = tuple(_make_call(nf, g == 0) for g, nf in enumerate(GROUPS))


def _group_prep(categories, embed_idx, f0, nf):
    cats = categories[:, f0:f0 + nf]
    cats3 = cats.T.reshape(nf, NW, BW).transpose(1, 0, 2)
    cats3 = cats3.reshape(NW, (nf * BW) // CHUNK, CHUNK)
    ei = embed_idx[f0:f0 + nf].astype(jnp.int32) - jnp.int32(f0 * V)
    ei_b = jnp.broadcast_to(ei.reshape(nf, 1), (nf, L))
    return cats3, ei_b


@jax.jit
def kernel(numbers, categories, W, bias, cat_params, embed_idx):
    nums3 = numbers.T.reshape(NUM, NW, BW).transpose(1, 0, 2)
    w_b = jnp.broadcast_to(W.reshape(NUM, 1), (NUM, L))
    bias_b = jnp.broadcast_to(bias.reshape(1, 1), (1, L)).reshape(L)
    acc = None
    f0 = 0
    for g, nf in enumerate(GROUPS):
        cats3, ei_b = _group_prep(categories, embed_idx, f0, nf)
        flat = cat_params[f0 * V:(f0 + nf) * V].reshape(nf * V)
        dummy = jnp.zeros(((nf * BW) // CHUNK, CHUNK), jnp.float32)
        p = _sc_calls[g](cats3, w_b, bias_b, ei_b, flat, dummy, nums3)
        acc = p if acc is None else acc + p
        f0 += nf
    return acc.reshape(B, OUT)
